# baseline (speedup 1.0000x reference)
import functools
import math

import jax
import jax.numpy as jnp
from jax import lax
from jax.experimental import pallas as pl
from jax.experimental.pallas import tpu as pltpu
from jax.experimental.pallas import tpu_sc as plsc

VOCAB = 100000
D_MODEL = 1024
SEQ_LEN = 2048
BATCH = 4

NTOK = BATCH * SEQ_LEN
NW = 32
B_PER_W = NTOK // NW
K = 32
NCHUNK = B_PER_W // K
NLANE = 16
NCOL = D_MODEL // NLANE


def _make_pe():
    position = jnp.arange(SEQ_LEN, dtype=jnp.float32)[:, None]
    div_term = jnp.exp(
        jnp.arange(0, D_MODEL, 2, dtype=jnp.float32)
        * (-math.log(10000.0) / D_MODEL))
    pe = jnp.zeros((SEQ_LEN, D_MODEL), dtype=jnp.float32)
    pe = pe.at[:, 0::2].set(jnp.sin(position * div_term))
    pe = pe.at[:, 1::2].set(jnp.cos(position * div_term))
    return pe


_mesh = plsc.VectorSubcoreMesh(core_axis_name="c", subcore_axis_name="s")


@functools.partial(
    pl.kernel,
    out_type=jax.ShapeDtypeStruct((NTOK, D_MODEL), jnp.float32),
    mesh=_mesh,
    scratch_types=[
        pltpu.VMEM((NCHUNK, K), jnp.int32),
        pltpu.VMEM((NCHUNK, K), jnp.int32),
        pltpu.VMEM((K, D_MODEL), jnp.float32),
        pltpu.VMEM((K, D_MODEL), jnp.float32),
        pltpu.VMEM((K, D_MODEL), jnp.float32),
        pltpu.SemaphoreType.DMA,
        pltpu.SemaphoreType.DMA,
    ],
)
def _emb_kernel(ids_hbm, sgids_hbm, emb_hbm, seg_hbm, pe_hbm, out_hbm,
                idx_v, sgid_v, emb_buf, seg_buf, pe_buf, sem0, sem1):
    wid = lax.axis_index("s") * 2 + lax.axis_index("c")
    base = wid * B_PER_W
    pos_base = lax.rem(base, SEQ_LEN)

    pltpu.sync_copy(ids_hbm.at[wid], idx_v)
    pltpu.sync_copy(sgids_hbm.at[wid], sgid_v)

    scale = jnp.float32(math.sqrt(D_MODEL))

    for j in range(NCHUNK):
        cp0 = pltpu.async_copy(emb_hbm.at[idx_v.at[j]], emb_buf, sem0)
        cp1 = pltpu.async_copy(seg_hbm.at[sgid_v.at[j]], seg_buf, sem1)
        pltpu.sync_copy(pe_hbm.at[pl.ds(pos_base + j * K, K)], pe_buf)
        cp0.wait()
        cp1.wait()

        def tbody(t, _):
            for c in range(NCOL):
                sl = pl.ds(c * NLANE, NLANE)
                emb_buf[t, sl] = (emb_buf[t, sl] * scale
                                  + pe_buf[t, sl] + seg_buf[t, sl])
            return _

        lax.fori_loop(0, K, tbody, None)

        pltpu.sync_copy(emb_buf, out_hbm.at[pl.ds(base + j * K, K)])


def kernel(input_ids, segment_ids, emb_table, seg_table):
    ids = input_ids.reshape(NW, NCHUNK, K).astype(jnp.int32)
    sgids = segment_ids.reshape(NW, NCHUNK, K).astype(jnp.int32)
    pe = _make_pe()
    out = _emb_kernel(ids, sgids, emb_table, seg_table, pe)
    return out.reshape(BATCH, SEQ_LEN, D_MODEL)
